# Initial kernel scaffold; baseline (speedup 1.0000x reference)
import jax
import jax.numpy as jnp
from jax.experimental import pallas as pl


def kernel(query, key, value, Wq, bq, Wk, bk, Wv, bv, Wo, bo, samples):
    raise NotImplementedError("write your pallas kernel here")


# speedup vs baseline: 154.2062x; 154.2062x over previous
import functools
import math

import jax
import jax.numpy as jnp
from jax import lax
from jax.experimental import pallas as pl
from jax.experimental.pallas import tpu as pltpu
from jax.experimental.pallas import tpu_sc as plsc

_N = 2048
_EMBED = 768
_HEADS = 12
_HEAD_DIM = _EMBED // _HEADS
_NUM_SAMPLES = 256
_SCALE = 1.0 / math.sqrt(float(_N))
_NEG = -1e30

_NW = 32
_ROWS_PER_W = _N // _NW
_CHUNK = 16
_G = _NUM_SAMPLES // 16


def _mask_body(samples_ref, mask_ref, idx_v, buf_v):
    wid = lax.axis_index("s") * 2 + lax.axis_index("c")
    base = wid * _ROWS_PER_W
    pltpu.sync_copy(
        samples_ref.at[pl.ds(base * _NUM_SAMPLES, _ROWS_PER_W * _NUM_SAMPLES)],
        idx_v,
    )
    neg = jnp.full((16,), _NEG, jnp.float32)
    zero = jnp.zeros((16,), jnp.float32)

    def fill(i, carry):
        buf_v[pl.ds(i * 16, 16)] = neg
        return carry

    lax.fori_loop(0, _CHUNK * _N // 16, fill, 0)

    n_chunks = _ROWS_PER_W // _CHUNK
    for c in range(n_chunks):
        def scatter_row(r, carry, _c=c, _val=zero):
            row_off = r * _N
            samp_off = (_c * _CHUNK + r) * _NUM_SAMPLES
            for g in range(_G):
                idx = idx_v[pl.ds(samp_off + g * 16, 16)] + row_off
                plsc.store_scatter(buf_v, [idx], _val)
            return carry

        lax.fori_loop(0, _CHUNK, scatter_row, 0)
        pltpu.sync_copy(
            buf_v, mask_ref.at[pl.ds((base + c * _CHUNK) * _N, _CHUNK * _N)]
        )

        if c + 1 < n_chunks:
            lax.fori_loop(
                0, _CHUNK, functools.partial(scatter_row, _val=neg), 0
            )


@functools.cache
def _get_mask_builder():
    return pl.kernel(
        _mask_body,
        out_type=jax.ShapeDtypeStruct((_N * _N,), jnp.float32),
        mesh=plsc.VectorSubcoreMesh(core_axis_name="c", subcore_axis_name="s"),
        scratch_types=[
            pltpu.VMEM((_ROWS_PER_W * _NUM_SAMPLES,), jnp.int32),
            pltpu.VMEM((_CHUNK * _N,), jnp.float32),
        ],
        compiler_params=pltpu.CompilerParams(
            needs_layout_passes=False, use_tc_tiling_on_sc=False
        ),
    )


def _build_mask(samples):
    return _get_mask_builder()(samples.reshape(-1)).reshape(_N, _N)

_BR = 512
_DN_T = (((1,), (1,)), ((), ()))
_DN_N = (((1,), (0,)), ((), ()))


def _kv_body(key_ref, value_ref, wk_ref, bk_ref, wv_ref, bv_ref, k_ref, v_ref):
    k_ref[0] = lax.dot_general(
        key_ref[...], wk_ref[0], _DN_T, preferred_element_type=jnp.float32
    ) + bk_ref[0]
    v_ref[0] = lax.dot_general(
        value_ref[...], wv_ref[0], _DN_T, preferred_element_type=jnp.float32
    ) + bv_ref[0]


def _attn_body(q_in_ref, wq_ref, bq_ref, k_ref, v_ref, mask_ref, wo_ref,
               bo_ref, out_ref):
    h = pl.program_id(1)
    q = lax.dot_general(
        q_in_ref[...], wq_ref[0], _DN_T, preferred_element_type=jnp.float32
    ) + bq_ref[0]
    logits = lax.dot_general(
        q, k_ref[0], _DN_T, preferred_element_type=jnp.float32
    ) * _SCALE + mask_ref[...]
    m = jnp.max(logits, axis=1, keepdims=True)
    e = jnp.exp(logits - m)
    s = jnp.sum(e, axis=1, keepdims=True)
    attn = lax.dot_general(
        e / s, v_ref[0], _DN_N, preferred_element_type=jnp.float32
    )
    o = lax.dot_general(
        attn, wo_ref[0], _DN_N, preferred_element_type=jnp.float32
    )

    @pl.when(h == 0)
    def _():
        out_ref[...] = o + bo_ref[...]

    @pl.when(h != 0)
    def _():
        out_ref[...] += o


def _head_spec():
    return pl.BlockSpec((1, _HEAD_DIM, _EMBED), lambda r, h: (h, 0, 0))


def _bias_spec():
    return pl.BlockSpec((1, 1, _HEAD_DIM), lambda r, h: (h, 0, 0))


_kv_proj = pl.pallas_call(
    _kv_body,
    grid=(_N // _BR, _HEADS),
    in_specs=[
        pl.BlockSpec((_BR, _EMBED), lambda r, h: (r, 0)),
        pl.BlockSpec((_BR, _EMBED), lambda r, h: (r, 0)),
        _head_spec(),
        _bias_spec(),
        _head_spec(),
        _bias_spec(),
    ],
    out_specs=[
        pl.BlockSpec((1, _BR, _HEAD_DIM), lambda r, h: (h, r, 0)),
        pl.BlockSpec((1, _BR, _HEAD_DIM), lambda r, h: (h, r, 0)),
    ],
    out_shape=[
        jax.ShapeDtypeStruct((_HEADS, _N, _HEAD_DIM), jnp.float32),
        jax.ShapeDtypeStruct((_HEADS, _N, _HEAD_DIM), jnp.float32),
    ],
)

_attn = pl.pallas_call(
    _attn_body,
    grid=(_N // _BR, _HEADS),
    in_specs=[
        pl.BlockSpec((_BR, _EMBED), lambda r, h: (r, 0)),
        _head_spec(),
        _bias_spec(),
        pl.BlockSpec((1, _N, _HEAD_DIM), lambda r, h: (h, 0, 0)),
        pl.BlockSpec((1, _N, _HEAD_DIM), lambda r, h: (h, 0, 0)),
        pl.BlockSpec((_BR, _N), lambda r, h: (r, 0)),
        _head_spec(),
        pl.BlockSpec((1, _EMBED), lambda r, h: (0, 0)),
    ],
    out_specs=pl.BlockSpec((_BR, _EMBED), lambda r, h: (r, 0)),
    out_shape=jax.ShapeDtypeStruct((_N, _EMBED), jnp.float32),
)


def kernel(query, key, value, Wq, bq, Wk, bk, Wv, bv, Wo, bo, samples):
    mask = _build_mask(samples)
    wk3 = Wk.reshape(_HEADS, _HEAD_DIM, _EMBED)
    wv3 = Wv.reshape(_HEADS, _HEAD_DIM, _EMBED)
    wq3 = Wq.reshape(_HEADS, _HEAD_DIM, _EMBED)
    wo3 = Wo.T.reshape(_HEADS, _HEAD_DIM, _EMBED)
    bk3 = bk.reshape(_HEADS, 1, _HEAD_DIM)
    bv3 = bv.reshape(_HEADS, 1, _HEAD_DIM)
    bq3 = bq.reshape(_HEADS, 1, _HEAD_DIM)
    k_h, v_h = _kv_proj(key, value, wk3, bk3, wv3, bv3)
    out = _attn(query, wq3, bq3, k_h, v_h, mask, wo3, bo.reshape(1, _EMBED))
    return out.reshape(_N, 1, _EMBED)
